# baseline (speedup 1.0000x reference)
import functools

import jax
import jax.numpy as jnp
from jax import lax
from jax.experimental import pallas as pl
from jax.experimental.pallas import tpu as pltpu
from jax.experimental.pallas import tpu_sc as plsc

D = 16
G = 8
FS = 8
NC = 2
NS = 16
NW = NC * NS
EPR = 128
U = 8
NZC = 32


def _feat_body(xp_ref, wj_ref, s_ref, o_ref, *, nj, nk):
    xp = xp_ref[...]
    sdot = jnp.dot(xp * xp, s_ref[...],
                   preferred_element_type=jnp.float32)
    w = wj_ref[...]
    m = jnp.dot(xp, w[0:G * FS], preferred_element_type=jnp.float32)
    for j in range(1, nj):
        m = jnp.minimum(
            m, jnp.dot(xp, w[j * G * FS:(j + 1) * G * FS],
                       preferred_element_type=jnp.float32))
    fd = m + sdot
    lane = lax.broadcasted_iota(jnp.int32, (1, 128), 1) % D
    o_ref[...] = jnp.where(lane == nk, 1.0,
                           jnp.where(lane < nk, fd, 0.0))


def _sc_body(edges_hbm, table_hbm, zeros_hbm, out_hbm,
             src_v, dst_v, rows_v, zbuf_v, agg_sh, gsem, ssem, isem,
             *, n_rows, s_rpt):
    c = lax.axis_index("c")
    s = lax.axis_index("s")
    wid = c * NS + s

    pltpu.sync_copy(zeros_hbm, zbuf_v)
    my0 = s * s_rpt

    def _zero(k, _):
        zch = s_rpt // NZC
        pltpu.sync_copy(zbuf_v, agg_sh.at[pl.ds(my0 + k * zch, zch)])
        return 0

    lax.fori_loop(0, NZC, _zero, 0)
    plsc.subcore_barrier()

    total_chunks = n_rows // U
    base_c = total_chunks // NW
    rem_c = total_chunks % NW
    n_chunks = base_c + jnp.where(wid < rem_c, 1, 0)
    start = (wid * base_c + jnp.minimum(wid, rem_c)) * U

    NB = 8
    A = 4

    def _idx_start(g, par):
        r0 = start + g * U
        pltpu.async_copy(edges_hbm.at[0, pl.ds(r0, U)], src_v.at[par], isem)
        pltpu.async_copy(edges_hbm.at[1, pl.ds(r0, U)], dst_v.at[par], isem)

    def _idx_wait(g, par):
        r0 = start + g * U
        pltpu.make_async_copy(
            edges_hbm.at[0, pl.ds(r0, U)], src_v.at[par], isem).wait()
        pltpu.make_async_copy(
            edges_hbm.at[1, pl.ds(r0, U)], dst_v.at[par], isem).wait()

    def _sca_wait_one(par):
        pltpu.make_async_copy(
            rows_v.at[0], agg_sh.at[dst_v.at[par, 0]], ssem).wait()

    @pl.when(n_chunks > 0)
    def _prologue():
        _idx_start(0, 0)

    def _chunk_body(g, par, first):
        _idx_wait(g, par)

        @pl.when(g + 1 < n_chunks)
        def _prefetch():
            _idx_start(g + 1, 1 - par)

        gat = {}
        for k in range(A):
            gat[k] = pltpu.async_copy(
                table_hbm.at[src_v.at[par, k]],
                rows_v.at[k % NB], gsem)
        for j in range(U):
            if not (first and j < NB - A):
                _sca_wait_one(par)
            if j + A < U:
                gat[j + A] = pltpu.async_copy(
                    table_hbm.at[src_v.at[par, j + A]],
                    rows_v.at[(j + A) % NB], gsem)
            gat[j].wait()
            pltpu.async_copy(
                rows_v.at[j % NB], agg_sh.at[dst_v.at[par, j]],
                ssem, add=True)

    @pl.when(n_chunks > 0)
    def _chunk0():
        _chunk_body(0, 0, True)

    def _chunk(g, _):
        _chunk_body(g, lax.rem(g, 2), False)
        return 0

    lax.fori_loop(1, n_chunks, _chunk, 0)

    def _drain(k, _):
        _sca_wait_one(0)
        return 0

    lax.fori_loop(0, jnp.minimum(NB - A, n_chunks * U), _drain, 0)

    def _tail(t, _):
        r = total_chunks * U + t
        pltpu.sync_copy(edges_hbm.at[0, pl.ds(r, 1)],
                        src_v.at[0].at[pl.ds(0, 1)])
        pltpu.sync_copy(edges_hbm.at[1, pl.ds(r, 1)],
                        dst_v.at[0].at[pl.ds(0, 1)])
        pltpu.async_copy(table_hbm.at[src_v.at[0, 0]],
                         rows_v.at[0], gsem).wait()
        pltpu.sync_copy(rows_v.at[0], agg_sh.at[dst_v.at[0, 0]], add=True)
        return 0

    n_tail = jnp.where(wid == NW - 1, n_rows - total_chunks * U, 0)
    lax.fori_loop(0, n_tail, _tail, 0)
    plsc.subcore_barrier()

    def _emit(k, _):
        zch = s_rpt // NZC
        r = my0 + k * zch
        pltpu.sync_copy(agg_sh.at[pl.ds(r, zch)], zbuf_v)
        pltpu.sync_copy(zbuf_v, out_hbm.at[c, pl.ds(r, zch)])
        return 0

    lax.fori_loop(0, NZC, _emit, 0)


def _final_body(tab_ref, p_ref, sel_ref, wtb_ref, st_ref, bp_ref, o_ref):
    a = p_ref[0] + p_ref[1]
    degb = jnp.dot(a, sel_ref[...],
                   preferred_element_type=jnp.float32)
    inv = 1.0 / jnp.maximum(degb, 1.0)
    h = 0.5 * (tab_ref[...] + a * inv) + st_ref[...]
    o = (jnp.dot(h, wtb_ref[...], preferred_element_type=jnp.float32)
         + bp_ref[...])
    o_ref[...] = o[0:o_ref.shape[0]]


def kernel(x, edge_index, templates, templates_features, W, b):
    n, f = x.shape
    e = edge_index.shape[1]
    nk, nj = templates_features.shape[0], templates_features.shape[1]
    nc = W.shape[0]
    s_rpt = NZC * -(-(n // NS + 1) // NZC)
    n_pad = NS * s_rpt
    npk = n_pad // G

    xg = jnp.concatenate([x, jnp.zeros((n_pad - n, f), x.dtype)])
    xpp = jnp.concatenate(
        [xg.reshape(npk, G, f),
         jnp.ones((npk, G, 1), jnp.float32),
         jnp.zeros((npk, G, FS - f - 1), jnp.float32)],
        axis=2).reshape(npk, G * FS)

    tf = templates_features
    tsq = jnp.sum(tf * tf, axis=2)
    w_small = jnp.zeros((nj, FS, D), jnp.float32)
    w_small = w_small.at[:, 0:f, 0:nk].set(-2.0 * jnp.transpose(tf, (1, 2, 0)))
    w_small = w_small.at[:, f, 0:nk].set(tsq.T - 1.0)
    eye = jnp.eye(G, dtype=jnp.float32)
    wj_big = jnp.stack(
        [jnp.kron(eye, w_small[j]) for j in range(nj)]
    ).reshape(nj * G * FS, G * D)
    s_big = jnp.kron(eye, jnp.ones((FS, D), jnp.float32))

    ow = 4
    sel_small = jnp.zeros((D, D), jnp.float32).at[nk, :].set(1.0)
    sel_big = jnp.kron(eye, sel_small)
    wt_small = jnp.zeros((D, ow), jnp.float32).at[0:nk, 0:nc].set(W.T)
    wt_big = jnp.kron(eye, wt_small)
    struct = jnp.mean(templates, axis=(1, 2))
    st_p = jnp.tile(jnp.pad(struct, (0, D - nk)), G).reshape(1, G * D)
    b_p = jnp.tile(jnp.pad(b, (0, ow - nc)), G).reshape(1, G * ow)

    blk = 3200
    grid1 = pl.cdiv(npk, blk)
    table_pk = pl.pallas_call(
        functools.partial(_feat_body, nj=nj, nk=nk),
        grid=(grid1,),
        in_specs=[
            pl.BlockSpec((blk, G * FS), lambda i: (i, 0)),
            pl.BlockSpec(wj_big.shape, lambda i: (0, 0)),
            pl.BlockSpec(s_big.shape, lambda i: (0, 0)),
        ],
        out_specs=pl.BlockSpec((blk, G * D), lambda i: (i, 0)),
        out_shape=jax.ShapeDtypeStruct((npk, G * D), jnp.float32),
    )(xpp, wj_big, s_big)
    table = table_pk.reshape(npk * G, D)

    if e % EPR:
        pad = EPR - e % EPR
        edge_index = jnp.concatenate(
            [edge_index,
             jnp.concatenate([jnp.zeros((1, pad), jnp.int32),
                              jnp.full((1, pad), n, jnp.int32)])], axis=1)
    n_rows = edge_index.shape[1] // EPR
    e3 = edge_index.reshape(2, n_rows, EPR)
    zeros_h = jnp.zeros((s_rpt // NZC, D), jnp.float32)

    mesh = plsc.VectorSubcoreMesh(core_axis_name="c", subcore_axis_name="s")
    parts = pl.kernel(
        functools.partial(_sc_body, n_rows=n_rows, s_rpt=s_rpt),
        out_type=jax.ShapeDtypeStruct((NC, n_pad, D), jnp.float32),
        mesh=mesh,
        scratch_types=[
            pltpu.VMEM((2, U, EPR), jnp.int32),
            pltpu.VMEM((2, U, EPR), jnp.int32),
            pltpu.VMEM((8, EPR, D), jnp.float32),
            pltpu.VMEM((s_rpt // NZC, D), jnp.float32),
            pltpu.VMEM_SHARED((n_pad, D), jnp.float32),
            pltpu.SemaphoreType.DMA,
            pltpu.SemaphoreType.DMA,
            pltpu.SemaphoreType.DMA,
        ],
        compiler_params=pltpu.CompilerParams(use_tc_tiling_on_sc=False),
    )(e3, table, zeros_h)
    parts_pk = parts.reshape(NC, npk, G * D)

    npo = n // G if n % G == 0 else npk
    out_pk = pl.pallas_call(
        _final_body,
        grid=(1,),
        in_specs=[
            pl.BlockSpec((npk, G * D), lambda i: (0, 0)),
            pl.BlockSpec((NC, npk, G * D), lambda i: (0, 0, 0)),
            pl.BlockSpec(sel_big.shape, lambda i: (0, 0)),
            pl.BlockSpec(wt_big.shape, lambda i: (0, 0)),
            pl.BlockSpec((1, G * D), lambda i: (0, 0)),
            pl.BlockSpec((1, G * ow), lambda i: (0, 0)),
        ],
        out_specs=pl.BlockSpec((npo, G * ow), lambda i: (0, 0)),
        out_shape=jax.ShapeDtypeStruct((npo, G * ow), jnp.float32),
    )(table_pk, parts_pk, sel_big, wt_big, st_p, b_p)

    return out_pk.reshape(npo * G, ow)[:n, :nc]


# speedup vs baseline: 1.0281x; 1.0281x over previous
import functools

import jax
import jax.numpy as jnp
from jax import lax
from jax.experimental import pallas as pl
from jax.experimental.pallas import tpu as pltpu
from jax.experimental.pallas import tpu_sc as plsc

D = 16
G = 8
FS = 8
NC = 2
NS = 16
NW = NC * NS
EPR = 128
U = 8
NZC = 32


def _feat_body(xp_ref, wj_ref, s_ref, o_ref, *, nj, nk):
    xp = xp_ref[...]
    sdot = jnp.dot(xp * xp, s_ref[...],
                   preferred_element_type=jnp.float32)
    w = wj_ref[...]
    m = jnp.dot(xp, w[0:G * FS], preferred_element_type=jnp.float32)
    for j in range(1, nj):
        m = jnp.minimum(
            m, jnp.dot(xp, w[j * G * FS:(j + 1) * G * FS],
                       preferred_element_type=jnp.float32))
    fd = m + sdot
    lane = lax.broadcasted_iota(jnp.int32, (1, 128), 1) % D
    o_ref[...] = jnp.where(lane == nk, 1.0,
                           jnp.where(lane < nk, fd, 0.0))


def _sc_body(edges_hbm, table_hbm, zeros_hbm, out_hbm,
             src_v, dst_v, rows_v, zbuf_v, agg_sh, gsem, ssem, isem,
             *, n_rows, s_rpt):
    c = lax.axis_index("c")
    s = lax.axis_index("s")
    wid = c * NS + s

    pltpu.sync_copy(zeros_hbm, zbuf_v)
    my0 = s * s_rpt

    def _zero(k, _):
        zch = s_rpt // NZC
        pltpu.sync_copy(zbuf_v, agg_sh.at[pl.ds(my0 + k * zch, zch)])
        return 0

    lax.fori_loop(0, NZC, _zero, 0)
    plsc.subcore_barrier()

    total_chunks = n_rows // U
    base_c = total_chunks // NW
    rem_c = total_chunks % NW
    n_chunks = base_c + jnp.where(wid < rem_c, 1, 0)
    start = (wid * base_c + jnp.minimum(wid, rem_c)) * U

    NB = 8
    A = 6

    def _idx_start(g, par):
        r0 = start + g * U
        pltpu.async_copy(edges_hbm.at[0, pl.ds(r0, U)], src_v.at[par], isem)
        pltpu.async_copy(edges_hbm.at[1, pl.ds(r0, U)], dst_v.at[par], isem)

    def _idx_wait(g, par):
        r0 = start + g * U
        pltpu.make_async_copy(
            edges_hbm.at[0, pl.ds(r0, U)], src_v.at[par], isem).wait()
        pltpu.make_async_copy(
            edges_hbm.at[1, pl.ds(r0, U)], dst_v.at[par], isem).wait()

    def _sca_wait_one(par):
        pltpu.make_async_copy(
            rows_v.at[0], agg_sh.at[dst_v.at[par, 0]], ssem).wait()

    @pl.when(n_chunks > 0)
    def _prologue():
        _idx_start(0, 0)

    def _chunk_body(g, par, first):
        _idx_wait(g, par)

        @pl.when(g + 1 < n_chunks)
        def _prefetch():
            _idx_start(g + 1, 1 - par)

        gat = {}
        for k in range(A):
            gat[k] = pltpu.async_copy(
                table_hbm.at[src_v.at[par, k]],
                rows_v.at[k % NB], gsem)
        for j in range(U):
            if not (first and j < NB - A):
                _sca_wait_one(par)
            if j + A < U:
                gat[j + A] = pltpu.async_copy(
                    table_hbm.at[src_v.at[par, j + A]],
                    rows_v.at[(j + A) % NB], gsem)
            gat[j].wait()
            pltpu.async_copy(
                rows_v.at[j % NB], agg_sh.at[dst_v.at[par, j]],
                ssem, add=True)

    @pl.when(n_chunks > 0)
    def _chunk0():
        _chunk_body(0, 0, True)

    def _chunk(g, _):
        _chunk_body(g, lax.rem(g, 2), False)
        return 0

    lax.fori_loop(1, n_chunks, _chunk, 0)

    def _drain(k, _):
        _sca_wait_one(0)
        return 0

    lax.fori_loop(0, jnp.minimum(NB - A, n_chunks * U), _drain, 0)

    def _tail(t, _):
        r = total_chunks * U + t
        pltpu.sync_copy(edges_hbm.at[0, pl.ds(r, 1)],
                        src_v.at[0].at[pl.ds(0, 1)])
        pltpu.sync_copy(edges_hbm.at[1, pl.ds(r, 1)],
                        dst_v.at[0].at[pl.ds(0, 1)])
        pltpu.async_copy(table_hbm.at[src_v.at[0, 0]],
                         rows_v.at[0], gsem).wait()
        pltpu.sync_copy(rows_v.at[0], agg_sh.at[dst_v.at[0, 0]], add=True)
        return 0

    n_tail = jnp.where(wid == NW - 1, n_rows - total_chunks * U, 0)
    lax.fori_loop(0, n_tail, _tail, 0)
    plsc.subcore_barrier()

    def _emit(k, _):
        zch = s_rpt // NZC
        r = my0 + k * zch
        pltpu.sync_copy(agg_sh.at[pl.ds(r, zch)], zbuf_v)
        pltpu.sync_copy(zbuf_v, out_hbm.at[c, pl.ds(r, zch)])
        return 0

    lax.fori_loop(0, NZC, _emit, 0)


def _final_body(tab_ref, p_ref, sel_ref, wtb_ref, st_ref, bp_ref, o_ref):
    a = p_ref[0] + p_ref[1]
    degb = jnp.dot(a, sel_ref[...],
                   preferred_element_type=jnp.float32)
    inv = 1.0 / jnp.maximum(degb, 1.0)
    h = 0.5 * (tab_ref[...] + a * inv) + st_ref[...]
    o = (jnp.dot(h, wtb_ref[...], preferred_element_type=jnp.float32)
         + bp_ref[...])
    o_ref[...] = o[0:o_ref.shape[0]]


def kernel(x, edge_index, templates, templates_features, W, b):
    n, f = x.shape
    e = edge_index.shape[1]
    nk, nj = templates_features.shape[0], templates_features.shape[1]
    nc = W.shape[0]
    s_rpt = NZC * -(-(n // NS + 1) // NZC)
    n_pad = NS * s_rpt
    npk = n_pad // G

    xg = jnp.concatenate([x, jnp.zeros((n_pad - n, f), x.dtype)])
    xpp = jnp.concatenate(
        [xg.reshape(npk, G, f),
         jnp.ones((npk, G, 1), jnp.float32),
         jnp.zeros((npk, G, FS - f - 1), jnp.float32)],
        axis=2).reshape(npk, G * FS)

    tf = templates_features
    tsq = jnp.sum(tf * tf, axis=2)
    w_small = jnp.zeros((nj, FS, D), jnp.float32)
    w_small = w_small.at[:, 0:f, 0:nk].set(-2.0 * jnp.transpose(tf, (1, 2, 0)))
    w_small = w_small.at[:, f, 0:nk].set(tsq.T - 1.0)
    eye = jnp.eye(G, dtype=jnp.float32)
    wj_big = jnp.stack(
        [jnp.kron(eye, w_small[j]) for j in range(nj)]
    ).reshape(nj * G * FS, G * D)
    s_big = jnp.kron(eye, jnp.ones((FS, D), jnp.float32))

    ow = 4
    sel_small = jnp.zeros((D, D), jnp.float32).at[nk, :].set(1.0)
    sel_big = jnp.kron(eye, sel_small)
    wt_small = jnp.zeros((D, ow), jnp.float32).at[0:nk, 0:nc].set(W.T)
    wt_big = jnp.kron(eye, wt_small)
    struct = jnp.mean(templates, axis=(1, 2))
    st_p = jnp.tile(jnp.pad(struct, (0, D - nk)), G).reshape(1, G * D)
    b_p = jnp.tile(jnp.pad(b, (0, ow - nc)), G).reshape(1, G * ow)

    blk = 3200
    grid1 = pl.cdiv(npk, blk)
    table_pk = pl.pallas_call(
        functools.partial(_feat_body, nj=nj, nk=nk),
        grid=(grid1,),
        in_specs=[
            pl.BlockSpec((blk, G * FS), lambda i: (i, 0)),
            pl.BlockSpec(wj_big.shape, lambda i: (0, 0)),
            pl.BlockSpec(s_big.shape, lambda i: (0, 0)),
        ],
        out_specs=pl.BlockSpec((blk, G * D), lambda i: (i, 0)),
        out_shape=jax.ShapeDtypeStruct((npk, G * D), jnp.float32),
    )(xpp, wj_big, s_big)
    table = table_pk.reshape(npk * G, D)

    if e % EPR:
        pad = EPR - e % EPR
        edge_index = jnp.concatenate(
            [edge_index,
             jnp.concatenate([jnp.zeros((1, pad), jnp.int32),
                              jnp.full((1, pad), n, jnp.int32)])], axis=1)
    n_rows = edge_index.shape[1] // EPR
    e3 = edge_index.reshape(2, n_rows, EPR)
    zeros_h = jnp.zeros((s_rpt // NZC, D), jnp.float32)

    mesh = plsc.VectorSubcoreMesh(core_axis_name="c", subcore_axis_name="s")
    parts = pl.kernel(
        functools.partial(_sc_body, n_rows=n_rows, s_rpt=s_rpt),
        out_type=jax.ShapeDtypeStruct((NC, n_pad, D), jnp.float32),
        mesh=mesh,
        scratch_types=[
            pltpu.VMEM((2, U, EPR), jnp.int32),
            pltpu.VMEM((2, U, EPR), jnp.int32),
            pltpu.VMEM((8, EPR, D), jnp.float32),
            pltpu.VMEM((s_rpt // NZC, D), jnp.float32),
            pltpu.VMEM_SHARED((n_pad, D), jnp.float32),
            pltpu.SemaphoreType.DMA,
            pltpu.SemaphoreType.DMA,
            pltpu.SemaphoreType.DMA,
        ],
        compiler_params=pltpu.CompilerParams(use_tc_tiling_on_sc=False),
    )(e3, table, zeros_h)
    parts_pk = parts.reshape(NC, npk, G * D)

    npo = n // G if n % G == 0 else npk
    out_pk = pl.pallas_call(
        _final_body,
        grid=(1,),
        in_specs=[
            pl.BlockSpec((npk, G * D), lambda i: (0, 0)),
            pl.BlockSpec((NC, npk, G * D), lambda i: (0, 0, 0)),
            pl.BlockSpec(sel_big.shape, lambda i: (0, 0)),
            pl.BlockSpec(wt_big.shape, lambda i: (0, 0)),
            pl.BlockSpec((1, G * D), lambda i: (0, 0)),
            pl.BlockSpec((1, G * ow), lambda i: (0, 0)),
        ],
        out_specs=pl.BlockSpec((npo, G * ow), lambda i: (0, 0)),
        out_shape=jax.ShapeDtypeStruct((npo, G * ow), jnp.float32),
    )(table_pk, parts_pk, sel_big, wt_big, st_p, b_p)

    return out_pk.reshape(npo * G, ow)[:n, :nc]


# speedup vs baseline: 1.0782x; 1.0487x over previous
import functools

import jax
import jax.numpy as jnp
from jax import lax
from jax.experimental import pallas as pl
from jax.experimental.pallas import tpu as pltpu
from jax.experimental.pallas import tpu_sc as plsc

D = 16
G = 8
FS = 8
NC = 2
NS = 16
NW = NC * NS
EPR = 128
U = 10
NZC = 32


def _feat_body(xp_ref, wj_ref, s_ref, o_ref, *, nj, nk):
    xp = xp_ref[...]
    sdot = jnp.dot(xp * xp, s_ref[...],
                   preferred_element_type=jnp.float32)
    w = wj_ref[...]
    m = jnp.dot(xp, w[0:G * FS], preferred_element_type=jnp.float32)
    for j in range(1, nj):
        m = jnp.minimum(
            m, jnp.dot(xp, w[j * G * FS:(j + 1) * G * FS],
                       preferred_element_type=jnp.float32))
    fd = m + sdot
    lane = lax.broadcasted_iota(jnp.int32, (1, 128), 1) % D
    o_ref[...] = jnp.where(lane == nk, 1.0,
                           jnp.where(lane < nk, fd, 0.0))


def _sc_body(edges_hbm, table_hbm, zeros_hbm, out_hbm,
             src_v, dst_v, rows_v, zbuf_v, agg_sh, gsem, ssem, isem,
             *, n_rows, s_rpt):
    c = lax.axis_index("c")
    s = lax.axis_index("s")
    wid = c * NS + s

    pltpu.sync_copy(zeros_hbm, zbuf_v)
    my0 = s * s_rpt

    def _zero(k, _):
        zch = s_rpt // NZC
        pltpu.sync_copy(zbuf_v, agg_sh.at[pl.ds(my0 + k * zch, zch)])
        return 0

    lax.fori_loop(0, NZC, _zero, 0)
    plsc.subcore_barrier()

    total_chunks = n_rows // U
    base_c = total_chunks // NW
    rem_c = total_chunks % NW
    n_chunks = base_c + jnp.where(wid < rem_c, 1, 0)
    start = (wid * base_c + jnp.minimum(wid, rem_c)) * U

    NB = 10
    A = 8

    def _idx_start(g, par):
        r0 = start + g * U
        pltpu.async_copy(edges_hbm.at[0, pl.ds(r0, U)], src_v.at[par], isem)
        pltpu.async_copy(edges_hbm.at[1, pl.ds(r0, U)], dst_v.at[par], isem)

    def _idx_wait(g, par):
        r0 = start + g * U
        pltpu.make_async_copy(
            edges_hbm.at[0, pl.ds(r0, U)], src_v.at[par], isem).wait()
        pltpu.make_async_copy(
            edges_hbm.at[1, pl.ds(r0, U)], dst_v.at[par], isem).wait()

    def _sca_wait_one(par):
        pltpu.make_async_copy(
            rows_v.at[0], agg_sh.at[dst_v.at[par, 0]], ssem).wait()

    @pl.when(n_chunks > 0)
    def _prologue():
        _idx_start(0, 0)

    def _chunk_body(g, par, first):
        _idx_wait(g, par)

        @pl.when(g + 1 < n_chunks)
        def _prefetch():
            _idx_start(g + 1, 1 - par)

        gat = {}
        for k in range(A):
            gat[k] = pltpu.async_copy(
                table_hbm.at[src_v.at[par, k]],
                rows_v.at[k % NB], gsem)
        for j in range(U):
            if not (first and j < NB - A):
                _sca_wait_one(par)
            if j + A < U:
                gat[j + A] = pltpu.async_copy(
                    table_hbm.at[src_v.at[par, j + A]],
                    rows_v.at[(j + A) % NB], gsem)
            gat[j].wait()
            pltpu.async_copy(
                rows_v.at[j % NB], agg_sh.at[dst_v.at[par, j]],
                ssem, add=True)

    @pl.when(n_chunks > 0)
    def _chunk0():
        _chunk_body(0, 0, True)

    def _chunk(g, _):
        _chunk_body(g, lax.rem(g, 2), False)
        return 0

    lax.fori_loop(1, n_chunks, _chunk, 0)

    def _drain(k, _):
        _sca_wait_one(0)
        return 0

    lax.fori_loop(0, jnp.minimum(NB - A, n_chunks * U), _drain, 0)

    def _tail(t, _):
        r = total_chunks * U + t
        pltpu.sync_copy(edges_hbm.at[0, pl.ds(r, 1)],
                        src_v.at[0].at[pl.ds(0, 1)])
        pltpu.sync_copy(edges_hbm.at[1, pl.ds(r, 1)],
                        dst_v.at[0].at[pl.ds(0, 1)])
        pltpu.async_copy(table_hbm.at[src_v.at[0, 0]],
                         rows_v.at[0], gsem).wait()
        pltpu.sync_copy(rows_v.at[0], agg_sh.at[dst_v.at[0, 0]], add=True)
        return 0

    n_tail = jnp.where(wid == NW - 1, n_rows - total_chunks * U, 0)
    lax.fori_loop(0, n_tail, _tail, 0)
    plsc.subcore_barrier()

    def _emit(k, _):
        zch = s_rpt // NZC
        r = my0 + k * zch
        pltpu.sync_copy(agg_sh.at[pl.ds(r, zch)], zbuf_v)
        pltpu.sync_copy(zbuf_v, out_hbm.at[c, pl.ds(r, zch)])
        return 0

    lax.fori_loop(0, NZC, _emit, 0)


def _final_body(tab_ref, p_ref, sel_ref, wtb_ref, st_ref, bp_ref, o_ref):
    a = p_ref[0] + p_ref[1]
    degb = jnp.dot(a, sel_ref[...],
                   preferred_element_type=jnp.float32)
    inv = 1.0 / jnp.maximum(degb, 1.0)
    h = 0.5 * (tab_ref[...] + a * inv) + st_ref[...]
    o = (jnp.dot(h, wtb_ref[...], preferred_element_type=jnp.float32)
         + bp_ref[...])
    o_ref[...] = o[0:o_ref.shape[0]]


def kernel(x, edge_index, templates, templates_features, W, b):
    n, f = x.shape
    e = edge_index.shape[1]
    nk, nj = templates_features.shape[0], templates_features.shape[1]
    nc = W.shape[0]
    s_rpt = NZC * -(-(n // NS + 1) // NZC)
    n_pad = NS * s_rpt
    npk = n_pad // G

    xg = jnp.concatenate([x, jnp.zeros((n_pad - n, f), x.dtype)])
    xpp = jnp.concatenate(
        [xg.reshape(npk, G, f),
         jnp.ones((npk, G, 1), jnp.float32),
         jnp.zeros((npk, G, FS - f - 1), jnp.float32)],
        axis=2).reshape(npk, G * FS)

    tf = templates_features
    tsq = jnp.sum(tf * tf, axis=2)
    w_small = jnp.zeros((nj, FS, D), jnp.float32)
    w_small = w_small.at[:, 0:f, 0:nk].set(-2.0 * jnp.transpose(tf, (1, 2, 0)))
    w_small = w_small.at[:, f, 0:nk].set(tsq.T - 1.0)
    eye = jnp.eye(G, dtype=jnp.float32)
    wj_big = jnp.stack(
        [jnp.kron(eye, w_small[j]) for j in range(nj)]
    ).reshape(nj * G * FS, G * D)
    s_big = jnp.kron(eye, jnp.ones((FS, D), jnp.float32))

    ow = 4
    sel_small = jnp.zeros((D, D), jnp.float32).at[nk, :].set(1.0)
    sel_big = jnp.kron(eye, sel_small)
    wt_small = jnp.zeros((D, ow), jnp.float32).at[0:nk, 0:nc].set(W.T)
    wt_big = jnp.kron(eye, wt_small)
    struct = jnp.mean(templates, axis=(1, 2))
    st_p = jnp.tile(jnp.pad(struct, (0, D - nk)), G).reshape(1, G * D)
    b_p = jnp.tile(jnp.pad(b, (0, ow - nc)), G).reshape(1, G * ow)

    blk = 3200
    grid1 = pl.cdiv(npk, blk)
    table_pk = pl.pallas_call(
        functools.partial(_feat_body, nj=nj, nk=nk),
        grid=(grid1,),
        in_specs=[
            pl.BlockSpec((blk, G * FS), lambda i: (i, 0)),
            pl.BlockSpec(wj_big.shape, lambda i: (0, 0)),
            pl.BlockSpec(s_big.shape, lambda i: (0, 0)),
        ],
        out_specs=pl.BlockSpec((blk, G * D), lambda i: (i, 0)),
        out_shape=jax.ShapeDtypeStruct((npk, G * D), jnp.float32),
    )(xpp, wj_big, s_big)
    table = table_pk.reshape(npk * G, D)

    if e % EPR:
        pad = EPR - e % EPR
        edge_index = jnp.concatenate(
            [edge_index,
             jnp.concatenate([jnp.zeros((1, pad), jnp.int32),
                              jnp.full((1, pad), n, jnp.int32)])], axis=1)
    n_rows = edge_index.shape[1] // EPR
    e3 = edge_index.reshape(2, n_rows, EPR)
    zeros_h = jnp.zeros((s_rpt // NZC, D), jnp.float32)

    mesh = plsc.VectorSubcoreMesh(core_axis_name="c", subcore_axis_name="s")
    parts = pl.kernel(
        functools.partial(_sc_body, n_rows=n_rows, s_rpt=s_rpt),
        out_type=jax.ShapeDtypeStruct((NC, n_pad, D), jnp.float32),
        mesh=mesh,
        scratch_types=[
            pltpu.VMEM((2, U, EPR), jnp.int32),
            pltpu.VMEM((2, U, EPR), jnp.int32),
            pltpu.VMEM((10, EPR, D), jnp.float32),
            pltpu.VMEM((s_rpt // NZC, D), jnp.float32),
            pltpu.VMEM_SHARED((n_pad, D), jnp.float32),
            pltpu.SemaphoreType.DMA,
            pltpu.SemaphoreType.DMA,
            pltpu.SemaphoreType.DMA,
        ],
        compiler_params=pltpu.CompilerParams(use_tc_tiling_on_sc=False),
    )(e3, table, zeros_h)
    parts_pk = parts.reshape(NC, npk, G * D)

    npo = n // G if n % G == 0 else npk
    out_pk = pl.pallas_call(
        _final_body,
        grid=(1,),
        in_specs=[
            pl.BlockSpec((npk, G * D), lambda i: (0, 0)),
            pl.BlockSpec((NC, npk, G * D), lambda i: (0, 0, 0)),
            pl.BlockSpec(sel_big.shape, lambda i: (0, 0)),
            pl.BlockSpec(wt_big.shape, lambda i: (0, 0)),
            pl.BlockSpec((1, G * D), lambda i: (0, 0)),
            pl.BlockSpec((1, G * ow), lambda i: (0, 0)),
        ],
        out_specs=pl.BlockSpec((npo, G * ow), lambda i: (0, 0)),
        out_shape=jax.ShapeDtypeStruct((npo, G * ow), jnp.float32),
    )(table_pk, parts_pk, sel_big, wt_big, st_p, b_p)

    return out_pk.reshape(npo * G, ow)[:n, :nc]
